# Initial kernel scaffold; baseline (speedup 1.0000x reference)
import jax
import jax.numpy as jnp
from jax.experimental import pallas as pl


def kernel(node_feat, edge_feat, edge_index, We1, be1, We2, be2, Wn1, bn1, Wn2, bn2):
    raise NotImplementedError("write your pallas kernel here")


# speedup vs baseline: 2.3616x; 2.3616x over previous
import functools

import jax
import jax.numpy as jnp
from jax import lax
from jax.experimental import pallas as pl
from jax.experimental.pallas import tpu as pltpu
from jax.experimental.pallas import tpu_sc as plsc

D = 128
NC = 2
NS = 16
NW = NC * NS
LANES = 16
CHUNK = 80


def _pre_body(nf_ref, ws_ref, wd_ref, p_ref, q_ref):
    nf = nf_ref[...]
    p_ref[...] = jnp.dot(nf, ws_ref[...], preferred_element_type=jnp.float32)
    q_ref[...] = jnp.dot(nf, wd_ref[...], preferred_element_type=jnp.float32)


def _edge_body(ef_ref, s_ref, w1_ref, b1_ref, w2_ref, b2_ref, ne_ref, eo_ref):
    ef = ef_ref[...]
    h = jnp.dot(ef, w1_ref[...], preferred_element_type=jnp.float32)
    h = jnp.maximum(h + s_ref[...] + b1_ref[...], 0.0)
    ne = jnp.dot(h, w2_ref[...], preferred_element_type=jnp.float32) + b2_ref[...]
    ne_ref[...] = ne
    eo_ref[...] = ne + ef


def _node_body(nf_ref, a0_ref, a1_ref, w1a_ref, w1b_ref, b1_ref, w2_ref,
               b2_ref, out_ref):
    nf = nf_ref[...]
    agg = a0_ref[...] + a1_ref[...]
    x = jnp.dot(nf, w1a_ref[...], preferred_element_type=jnp.float32)
    x = x + jnp.dot(agg, w1b_ref[...], preferred_element_type=jnp.float32)
    h = jnp.maximum(x + b1_ref[...], 0.0)
    out_ref[...] = (jnp.dot(h, w2_ref[...], preferred_element_type=jnp.float32)
                    + b2_ref[...] + nf)


def _gather_body(e_total, p_hbm, q_hbm, src_hbm, dst_hbm, s_hbm,
                 idx_a, idx_b, buf_a, buf_b, sem_a, sem_b):
    w = lax.axis_index("c") * NS + lax.axis_index("s")
    e_per_w = e_total // NW
    n_chunks = e_per_w // CHUNK
    base_w = w * e_per_w

    def chunk_body(j, carry):
        base = base_w + j * CHUNK
        pltpu.sync_copy(src_hbm.at[pl.ds(base, CHUNK)], idx_a)
        pltpu.sync_copy(dst_hbm.at[pl.ds(base, CHUNK)], idx_b)
        cp_a = pltpu.async_copy(p_hbm.at[idx_a], buf_a, sem_a)
        cp_b = pltpu.async_copy(q_hbm.at[idx_b], buf_b, sem_b)
        cp_a.wait()
        cp_b.wait()

        def add_row(r, c2):
            for c in range(D // LANES):
                sl = pl.ds(c * LANES, LANES)
                buf_a[r, sl] = buf_a[r, sl] + buf_b[r, sl]
            return c2

        lax.fori_loop(0, CHUNK, add_row, 0, unroll=2)
        pltpu.sync_copy(buf_a, s_hbm.at[pl.ds(base, CHUNK)])
        return carry

    lax.fori_loop(0, n_chunks, chunk_body, 0)


def _scatter_body(n_nodes, e_total, ne_hbm, dst_hbm, agg_hbm,
                  idx_v, buf, zbuf, acc):
    cid = lax.axis_index("c")
    sid = lax.axis_index("s")
    rows_per_tile = (n_nodes // NS) // 8 * 8
    tail_rows = n_nodes - NS * rows_per_tile
    zrows = zbuf.shape[0]

    def zero_row(r, carry):
        for c in range(D // LANES):
            zbuf[r, pl.ds(c * LANES, LANES)] = jnp.zeros((LANES,), jnp.float32)
        return carry

    lax.fori_loop(0, zrows, zero_row, 0)
    for b in range(rows_per_tile // zrows):
        pltpu.sync_copy(zbuf, acc.at[pl.ds(sid * rows_per_tile + b * zrows, zrows)])

    @pl.when(sid == 0)
    def _zero_tail():
        pltpu.sync_copy(zbuf.at[pl.ds(0, tail_rows)],
                        acc.at[pl.ds(NS * rows_per_tile, tail_rows)])

    plsc.subcore_barrier()

    w = cid * NS + sid
    e_per_w = e_total // NW
    n_chunks = e_per_w // CHUNK
    base_w = w * e_per_w

    def chunk_body(j, carry):
        base = base_w + j * CHUNK
        pltpu.sync_copy(dst_hbm.at[pl.ds(base, CHUNK)], idx_v)
        pltpu.sync_copy(ne_hbm.at[pl.ds(base, CHUNK)], buf)
        pltpu.sync_copy(buf, acc.at[idx_v], add=True)
        return carry

    lax.fori_loop(0, n_chunks, chunk_body, 0)
    plsc.subcore_barrier()
    r0 = sid * rows_per_tile
    pltpu.sync_copy(acc.at[pl.ds(r0, rows_per_tile)],
                    agg_hbm.at[cid, pl.ds(r0, rows_per_tile)])

    @pl.when(sid == 0)
    def _write_tail():
        pltpu.sync_copy(acc.at[pl.ds(NS * rows_per_tile, tail_rows)],
                        agg_hbm.at[cid, pl.ds(NS * rows_per_tile, tail_rows)])


def _tc_pre(node_feat, w_src, w_dst, block):
    n = node_feat.shape[0]
    sds = jax.ShapeDtypeStruct((n, D), jnp.float32)
    return pl.pallas_call(
        _pre_body,
        grid=(n // block,),
        in_specs=[
            pl.BlockSpec((block, D), lambda i: (i, 0)),
            pl.BlockSpec((D, D), lambda i: (0, 0)),
            pl.BlockSpec((D, D), lambda i: (0, 0)),
        ],
        out_specs=[pl.BlockSpec((block, D), lambda i: (i, 0))] * 2,
        out_shape=[sds, sds],
    )(node_feat, w_src, w_dst)


def _tc_edge(edge_feat, s, w1, b1, w2, b2, block):
    e = edge_feat.shape[0]
    sds = jax.ShapeDtypeStruct((e, D), jnp.float32)
    return pl.pallas_call(
        _edge_body,
        grid=(e // block,),
        in_specs=[
            pl.BlockSpec((block, D), lambda i: (i, 0)),
            pl.BlockSpec((block, D), lambda i: (i, 0)),
            pl.BlockSpec((D, D), lambda i: (0, 0)),
            pl.BlockSpec((1, D), lambda i: (0, 0)),
            pl.BlockSpec((D, D), lambda i: (0, 0)),
            pl.BlockSpec((1, D), lambda i: (0, 0)),
        ],
        out_specs=[pl.BlockSpec((block, D), lambda i: (i, 0))] * 2,
        out_shape=[sds, sds],
    )(edge_feat, s, w1, b1, w2, b2)


def _tc_node(node_feat, a0, a1, w1a, w1b, b1, w2, b2, block):
    n = node_feat.shape[0]
    return pl.pallas_call(
        _node_body,
        grid=(n // block,),
        in_specs=[
            pl.BlockSpec((block, D), lambda i: (i, 0)),
            pl.BlockSpec((block, D), lambda i: (i, 0)),
            pl.BlockSpec((block, D), lambda i: (i, 0)),
            pl.BlockSpec((D, D), lambda i: (0, 0)),
            pl.BlockSpec((D, D), lambda i: (0, 0)),
            pl.BlockSpec((1, D), lambda i: (0, 0)),
            pl.BlockSpec((D, D), lambda i: (0, 0)),
            pl.BlockSpec((1, D), lambda i: (0, 0)),
        ],
        out_specs=pl.BlockSpec((block, D), lambda i: (i, 0)),
        out_shape=jax.ShapeDtypeStruct((n, D), jnp.float32),
    )(node_feat, a0, a1, w1a, w1b, b1, w2, b2)


def _sc_gather(p, q, src, dst):
    e = src.shape[0]
    mesh = plsc.VectorSubcoreMesh(core_axis_name="c", subcore_axis_name="s",
                                  num_cores=NC, num_subcores=NS)
    k = pl.kernel(
        functools.partial(_gather_body, e),
        out_type=jax.ShapeDtypeStruct((e, D), jnp.float32),
        mesh=mesh,
        scratch_types=[
            pltpu.VMEM((CHUNK,), jnp.int32),
            pltpu.VMEM((CHUNK,), jnp.int32),
            pltpu.VMEM((CHUNK, D), jnp.float32),
            pltpu.VMEM((CHUNK, D), jnp.float32),
            pltpu.SemaphoreType.DMA,
            pltpu.SemaphoreType.DMA,
        ],
    )
    return k(p, q, src, dst)


def _sc_scatter(ne, dst, n_nodes):
    e = dst.shape[0]
    zrows = ((n_nodes // NS) // 8 * 8) // 3
    mesh = plsc.VectorSubcoreMesh(core_axis_name="c", subcore_axis_name="s",
                                  num_cores=NC, num_subcores=NS)
    k = pl.kernel(
        functools.partial(_scatter_body, n_nodes, e),
        out_type=jax.ShapeDtypeStruct((NC, n_nodes, D), jnp.float32),
        mesh=mesh,
        scratch_types=[
            pltpu.VMEM((CHUNK,), jnp.int32),
            pltpu.VMEM((CHUNK, D), jnp.float32),
            pltpu.VMEM((zrows, D), jnp.float32),
            pltpu.VMEM_SHARED((n_nodes, D), jnp.float32),
        ],
    )
    return k(ne, dst)


def kernel(node_feat, edge_feat, edge_index, We1, be1, We2, be2, Wn1, bn1,
           Wn2, bn2):
    n = node_feat.shape[0]
    src = edge_index[0].astype(jnp.int32)
    dst = edge_index[1].astype(jnp.int32)
    w_e = We1[:D]
    w_s = We1[D:2 * D]
    w_d = We1[2 * D:]
    w1a = Wn1[:D]
    w1b = Wn1[D:]
    be1_2d = be1.reshape(1, D)
    be2_2d = be2.reshape(1, D)
    bn1_2d = bn1.reshape(1, D)
    bn2_2d = bn2.reshape(1, D)

    p, q = _tc_pre(node_feat, w_s, w_d, block=2000)
    s = _sc_gather(p, q, src, dst)
    ne, edge_out = _tc_edge(edge_feat, s, w_e, be1_2d, We2, be2_2d, block=2000)
    aggp = _sc_scatter(ne, dst, n)
    node_out = _tc_node(node_feat, aggp[0], aggp[1], w1a, w1b, bn1_2d, Wn2,
                        bn2_2d, block=2000)
    return node_out, edge_out


# speedup vs baseline: 3.2649x; 1.3825x over previous
import functools

import jax
import jax.numpy as jnp
from jax import lax
from jax.experimental import pallas as pl
from jax.experimental.pallas import tpu as pltpu
from jax.experimental.pallas import tpu_sc as plsc

D = 128
NC = 2
NS = 16
NW = NC * NS
LANES = 16
CHUNK = 80


def _pre_body(nf_ref, ws_ref, wd_ref, p_ref, q_ref):
    nf = nf_ref[...]
    p_ref[...] = jnp.dot(nf, ws_ref[...], preferred_element_type=jnp.float32)
    q_ref[...] = jnp.dot(nf, wd_ref[...], preferred_element_type=jnp.float32)


def _edge_body(ef_ref, s_ref, w1_ref, b1_ref, w2_ref, b2_ref, ne_ref, eo_ref):
    ef = ef_ref[...]
    h = jnp.dot(ef, w1_ref[...], preferred_element_type=jnp.float32)
    h = jnp.maximum(h + s_ref[...] + b1_ref[...], 0.0)
    ne = jnp.dot(h, w2_ref[...], preferred_element_type=jnp.float32) + b2_ref[...]
    ne_ref[...] = ne
    eo_ref[...] = ne + ef


def _node_body(nf_ref, a0_ref, a1_ref, w1a_ref, w1b_ref, b1_ref, w2_ref,
               b2_ref, out_ref):
    nf = nf_ref[...]
    agg = a0_ref[...] + a1_ref[...]
    x = jnp.dot(nf, w1a_ref[...], preferred_element_type=jnp.float32)
    x = x + jnp.dot(agg, w1b_ref[...], preferred_element_type=jnp.float32)
    h = jnp.maximum(x + b1_ref[...], 0.0)
    out_ref[...] = (jnp.dot(h, w2_ref[...], preferred_element_type=jnp.float32)
                    + b2_ref[...] + nf)


def _gather_body(e_total, p_hbm, q_hbm, src3_hbm, dst3_hbm, s_hbm,
                 idx_s, idx_d, buf_a, buf_b, sem_a, sem_b, sem_o):
    w = lax.axis_index("c") * NS + lax.axis_index("s")
    e_per_w = e_total // NW
    n_chunks = e_per_w // CHUNK
    base_w = w * e_per_w

    pltpu.sync_copy(src3_hbm.at[w], idx_s)
    pltpu.sync_copy(dst3_hbm.at[w], idx_d)

    def issue(j, p):
        pltpu.async_copy(p_hbm.at[idx_s.at[j]], buf_a[p], sem_a[p])
        pltpu.async_copy(q_hbm.at[idx_d.at[j]], buf_b[p], sem_b[p])

    def process(j, p):
        pltpu.make_async_copy(p_hbm.at[idx_s.at[j]], buf_a[p], sem_a[p]).wait()
        pltpu.make_async_copy(q_hbm.at[idx_d.at[j]], buf_b[p], sem_b[p]).wait()

        def add_row(r, c2):
            for c in range(D // LANES):
                sl = pl.ds(c * LANES, LANES)
                buf_a[p][r, sl] = buf_a[p][r, sl] + buf_b[p][r, sl]
            return c2

        lax.fori_loop(0, CHUNK, add_row, 0, unroll=2)
        pltpu.async_copy(buf_a[p], s_hbm.at[pl.ds(base_w + j * CHUNK, CHUNK)],
                         sem_o[p])

    def drain_out(j, p):
        pltpu.make_async_copy(buf_a[p], s_hbm.at[pl.ds(base_w + j * CHUNK, CHUNK)],
                              sem_o[p]).wait()

    issue(0, 0)
    issue(1, 1)

    def pair_body(u, carry):
        process(2 * u, 0)
        process(2 * u + 1, 1)
        drain_out(2 * u, 0)
        issue(2 * u + 2, 0)
        drain_out(2 * u + 1, 1)
        issue(2 * u + 3, 1)
        return carry

    n_pairs = (n_chunks - 3) // 2
    lax.fori_loop(0, n_pairs, pair_body, 0)
    j0 = 2 * n_pairs
    process(j0, 0)
    process(j0 + 1, 1)
    drain_out(j0, 0)
    issue(j0 + 2, 0)
    process(j0 + 2, 0)
    drain_out(j0 + 1, 1)
    drain_out(j0 + 2, 0)


def _scatter_body(n_nodes, e_total, ne_hbm, dst3_hbm, agg_hbm,
                  idx_v, buf, zbuf, acc, sem):
    cid = lax.axis_index("c")
    sid = lax.axis_index("s")
    rows_per_tile = (n_nodes // NS) // 8 * 8
    tail_rows = n_nodes - NS * rows_per_tile
    zrows = zbuf.shape[0]

    def zero_row(r, carry):
        for c in range(D // LANES):
            zbuf[r, pl.ds(c * LANES, LANES)] = jnp.zeros((LANES,), jnp.float32)
        return carry

    lax.fori_loop(0, zrows, zero_row, 0)
    for b in range(rows_per_tile // zrows):
        pltpu.sync_copy(zbuf, acc.at[pl.ds(sid * rows_per_tile + b * zrows, zrows)])

    @pl.when(sid == 0)
    def _zero_tail():
        pltpu.sync_copy(zbuf.at[pl.ds(0, tail_rows)],
                        acc.at[pl.ds(NS * rows_per_tile, tail_rows)])

    plsc.subcore_barrier()

    w = cid * NS + sid
    e_per_w = e_total // NW
    n_chunks = e_per_w // CHUNK
    base_w = w * e_per_w
    pltpu.sync_copy(dst3_hbm.at[w], idx_v)

    def issue(j, p):
        pltpu.async_copy(ne_hbm.at[pl.ds(base_w + j * CHUNK, CHUNK)], buf[p],
                         sem[p])

    def process(j, p):
        pltpu.make_async_copy(ne_hbm.at[pl.ds(base_w + j * CHUNK, CHUNK)],
                              buf[p], sem[p]).wait()
        pltpu.sync_copy(buf[p], acc.at[idx_v.at[j]], add=True)

    issue(0, 0)
    issue(1, 1)

    def pair_body(u, carry):
        process(2 * u, 0)
        issue(2 * u + 2, 0)
        process(2 * u + 1, 1)
        issue(2 * u + 3, 1)
        return carry

    n_pairs = (n_chunks - 3) // 2
    lax.fori_loop(0, n_pairs, pair_body, 0)
    j0 = 2 * n_pairs
    process(j0, 0)
    issue(j0 + 2, 0)
    process(j0 + 1, 1)
    process(j0 + 2, 0)
    plsc.subcore_barrier()
    r0 = sid * rows_per_tile
    pltpu.sync_copy(acc.at[pl.ds(r0, rows_per_tile)],
                    agg_hbm.at[cid, pl.ds(r0, rows_per_tile)])

    @pl.when(sid == 0)
    def _write_tail():
        pltpu.sync_copy(acc.at[pl.ds(NS * rows_per_tile, tail_rows)],
                        agg_hbm.at[cid, pl.ds(NS * rows_per_tile, tail_rows)])


def _tc_pre(node_feat, w_src, w_dst, block):
    n = node_feat.shape[0]
    sds = jax.ShapeDtypeStruct((n, D), jnp.float32)
    return pl.pallas_call(
        _pre_body,
        grid=(n // block,),
        in_specs=[
            pl.BlockSpec((block, D), lambda i: (i, 0)),
            pl.BlockSpec((D, D), lambda i: (0, 0)),
            pl.BlockSpec((D, D), lambda i: (0, 0)),
        ],
        out_specs=[pl.BlockSpec((block, D), lambda i: (i, 0))] * 2,
        out_shape=[sds, sds],
    )(node_feat, w_src, w_dst)


def _tc_edge(edge_feat, s, w1, b1, w2, b2, block):
    e = edge_feat.shape[0]
    sds = jax.ShapeDtypeStruct((e, D), jnp.float32)
    return pl.pallas_call(
        _edge_body,
        grid=(e // block,),
        in_specs=[
            pl.BlockSpec((block, D), lambda i: (i, 0)),
            pl.BlockSpec((block, D), lambda i: (i, 0)),
            pl.BlockSpec((D, D), lambda i: (0, 0)),
            pl.BlockSpec((1, D), lambda i: (0, 0)),
            pl.BlockSpec((D, D), lambda i: (0, 0)),
            pl.BlockSpec((1, D), lambda i: (0, 0)),
        ],
        out_specs=[pl.BlockSpec((block, D), lambda i: (i, 0))] * 2,
        out_shape=[sds, sds],
    )(edge_feat, s, w1, b1, w2, b2)


def _tc_node(node_feat, a0, a1, w1a, w1b, b1, w2, b2, block):
    n = node_feat.shape[0]
    return pl.pallas_call(
        _node_body,
        grid=(n // block,),
        in_specs=[
            pl.BlockSpec((block, D), lambda i: (i, 0)),
            pl.BlockSpec((block, D), lambda i: (i, 0)),
            pl.BlockSpec((block, D), lambda i: (i, 0)),
            pl.BlockSpec((D, D), lambda i: (0, 0)),
            pl.BlockSpec((D, D), lambda i: (0, 0)),
            pl.BlockSpec((1, D), lambda i: (0, 0)),
            pl.BlockSpec((D, D), lambda i: (0, 0)),
            pl.BlockSpec((1, D), lambda i: (0, 0)),
        ],
        out_specs=pl.BlockSpec((block, D), lambda i: (i, 0)),
        out_shape=jax.ShapeDtypeStruct((n, D), jnp.float32),
    )(node_feat, a0, a1, w1a, w1b, b1, w2, b2)


def _sc_gather(p, q, src3, dst3):
    e = src3.size
    n_chunks = (e // NW) // CHUNK
    mesh = plsc.VectorSubcoreMesh(core_axis_name="c", subcore_axis_name="s",
                                  num_cores=NC, num_subcores=NS)
    k = pl.kernel(
        functools.partial(_gather_body, e),
        out_type=jax.ShapeDtypeStruct((e, D), jnp.float32),
        mesh=mesh,
        scratch_types=[
            pltpu.VMEM((n_chunks, CHUNK), jnp.int32),
            pltpu.VMEM((n_chunks, CHUNK), jnp.int32),
            [pltpu.VMEM((CHUNK, D), jnp.float32)] * 2,
            [pltpu.VMEM((CHUNK, D), jnp.float32)] * 2,
            [pltpu.SemaphoreType.DMA] * 2,
            [pltpu.SemaphoreType.DMA] * 2,
            [pltpu.SemaphoreType.DMA] * 2,
        ],
    )
    return k(p, q, src3, dst3)


def _sc_scatter(ne, dst3, n_nodes):
    e = dst3.size
    n_chunks = (e // NW) // CHUNK
    zrows = ((n_nodes // NS) // 8 * 8) // 6
    mesh = plsc.VectorSubcoreMesh(core_axis_name="c", subcore_axis_name="s",
                                  num_cores=NC, num_subcores=NS)
    k = pl.kernel(
        functools.partial(_scatter_body, n_nodes, e),
        out_type=jax.ShapeDtypeStruct((NC, n_nodes, D), jnp.float32),
        mesh=mesh,
        scratch_types=[
            pltpu.VMEM((n_chunks, CHUNK), jnp.int32),
            [pltpu.VMEM((CHUNK, D), jnp.float32)] * 2,
            pltpu.VMEM((zrows, D), jnp.float32),
            pltpu.VMEM_SHARED((n_nodes, D), jnp.float32),
            [pltpu.SemaphoreType.DMA] * 2,
        ],
    )
    return k(ne, dst3)


def kernel(node_feat, edge_feat, edge_index, We1, be1, We2, be2, Wn1, bn1,
           Wn2, bn2):
    n = node_feat.shape[0]
    e = edge_feat.shape[0]
    n_chunks = (e // NW) // CHUNK
    src3 = edge_index[0].astype(jnp.int32).reshape(NW, n_chunks, CHUNK)
    dst3 = edge_index[1].astype(jnp.int32).reshape(NW, n_chunks, CHUNK)
    w_e = We1[:D]
    w_s = We1[D:2 * D]
    w_d = We1[2 * D:]
    w1a = Wn1[:D]
    w1b = Wn1[D:]
    be1_2d = be1.reshape(1, D)
    be2_2d = be2.reshape(1, D)
    bn1_2d = bn1.reshape(1, D)
    bn2_2d = bn2.reshape(1, D)

    p, q = _tc_pre(node_feat, w_s, w_d, block=2000)
    s = _sc_gather(p, q, src3, dst3)
    ne, edge_out = _tc_edge(edge_feat, s, w_e, be1_2d, We2, be2_2d, block=2000)
    aggp = _sc_scatter(ne, dst3, n)
    node_out = _tc_node(node_feat, aggp[0], aggp[1], w1a, w1b, bn1_2d, Wn2,
                        bn2_2d, block=2000)
    return node_out, edge_out


# speedup vs baseline: 4.8617x; 1.4891x over previous
import functools

import jax
import jax.numpy as jnp
from jax import lax
from jax.experimental import pallas as pl
from jax.experimental.pallas import tpu as pltpu
from jax.experimental.pallas import tpu_sc as plsc

D = 128
NC = 2
NS = 16
NW = NC * NS
LANES = 16
CHUNK = 80


def _pre_body(nf_ref, ws_ref, wd_ref, p_ref, q_ref):
    nf = nf_ref[...]
    p_ref[...] = jnp.dot(nf, ws_ref[...], preferred_element_type=jnp.float32)
    q_ref[...] = jnp.dot(nf, wd_ref[...], preferred_element_type=jnp.float32)


def _edge_body(ef_ref, s_ref, w1_ref, b1_ref, w2_ref, b2_ref, ne_ref, eo_ref):
    ef = ef_ref[...]
    h = jnp.dot(ef, w1_ref[...], preferred_element_type=jnp.float32)
    h = jnp.maximum(h + s_ref[...] + b1_ref[...], 0.0)
    ne = jnp.dot(h, w2_ref[...], preferred_element_type=jnp.float32) + b2_ref[...]
    ne_ref[...] = ne
    eo_ref[...] = ne + ef


def _node_body(nf_ref, a0_ref, a1_ref, w1a_ref, w1b_ref, b1_ref, w2_ref,
               b2_ref, out_ref):
    nf = nf_ref[...]
    agg = a0_ref[...] + a1_ref[...]
    x = jnp.dot(nf, w1a_ref[...], preferred_element_type=jnp.float32)
    x = x + jnp.dot(agg, w1b_ref[...], preferred_element_type=jnp.float32)
    h = jnp.maximum(x + b1_ref[...], 0.0)
    out_ref[...] = (jnp.dot(h, w2_ref[...], preferred_element_type=jnp.float32)
                    + b2_ref[...] + nf)


def _gather_body(e_total, p_hbm, q_hbm, src3_hbm, dst3_hbm, s_hbm,
                 idx_s, idx_d, buf_a, buf_b, sem_a, sem_b, sem_o):
    w = lax.axis_index("c") * NS + lax.axis_index("s")
    e_per_w = e_total // NW
    n_chunks = e_per_w // CHUNK
    base_w = w * e_per_w

    pltpu.sync_copy(src3_hbm.at[w], idx_s)
    pltpu.sync_copy(dst3_hbm.at[w], idx_d)

    def issue(j, p):
        pltpu.async_copy(p_hbm.at[idx_s.at[j]], buf_a[p], sem_a[p])
        pltpu.async_copy(q_hbm.at[idx_d.at[j]], buf_b[p], sem_b[p])

    def process(j, p):
        pltpu.make_async_copy(p_hbm.at[idx_s.at[j]], buf_a[p], sem_a[p]).wait()
        pltpu.make_async_copy(q_hbm.at[idx_d.at[j]], buf_b[p], sem_b[p]).wait()

        def add_row(r, c2):
            for c in range(D // LANES):
                sl = pl.ds(c * LANES, LANES)
                buf_a[p][r, sl] = buf_a[p][r, sl] + buf_b[p][r, sl]
            return c2

        @functools.partial(plsc.parallel_loop, 0, CHUNK, unroll=4)
        def add_row(r):
            for c in range(D // LANES):
                sl = pl.ds(c * LANES, LANES)
                buf_a[p][r, sl] = buf_a[p][r, sl] + buf_b[p][r, sl]

        pltpu.async_copy(buf_a[p], s_hbm.at[pl.ds(base_w + j * CHUNK, CHUNK)],
                         sem_o[p])

    def drain_out(j, p):
        pltpu.make_async_copy(buf_a[p], s_hbm.at[pl.ds(base_w + j * CHUNK, CHUNK)],
                              sem_o[p]).wait()

    issue(0, 0)
    issue(1, 1)

    def pair_body(u, carry):
        process(2 * u, 0)
        process(2 * u + 1, 1)
        drain_out(2 * u, 0)
        issue(2 * u + 2, 0)
        drain_out(2 * u + 1, 1)
        issue(2 * u + 3, 1)
        return carry

    n_pairs = (n_chunks - 3) // 2
    lax.fori_loop(0, n_pairs, pair_body, 0)
    j0 = 2 * n_pairs
    process(j0, 0)
    process(j0 + 1, 1)
    drain_out(j0, 0)
    issue(j0 + 2, 0)
    process(j0 + 2, 0)
    drain_out(j0 + 1, 1)
    drain_out(j0 + 2, 0)


def _scatter_body(n_nodes, e_total, ne_hbm, dst3_hbm, agg_hbm,
                  idx_v, buf, zbuf, acc, sem):
    cid = lax.axis_index("c")
    sid = lax.axis_index("s")
    rows_per_tile = (n_nodes // NS) // 8 * 8
    tail_rows = n_nodes - NS * rows_per_tile
    zrows = zbuf.shape[0]

    def zero_row(r, carry):
        for c in range(D // LANES):
            zbuf[r, pl.ds(c * LANES, LANES)] = jnp.zeros((LANES,), jnp.float32)
        return carry

    lax.fori_loop(0, zrows, zero_row, 0)
    for b in range(rows_per_tile // zrows):
        pltpu.sync_copy(zbuf, acc.at[pl.ds(sid * rows_per_tile + b * zrows, zrows)])

    @pl.when(sid == 0)
    def _zero_tail():
        pltpu.sync_copy(zbuf.at[pl.ds(0, tail_rows)],
                        acc.at[pl.ds(NS * rows_per_tile, tail_rows)])

    plsc.subcore_barrier()

    w = cid * NS + sid
    e_per_w = e_total // NW
    n_chunks = e_per_w // CHUNK
    base_w = w * e_per_w
    pltpu.sync_copy(dst3_hbm.at[w], idx_v)

    def issue(j, p):
        pltpu.async_copy(ne_hbm.at[pl.ds(base_w + j * CHUNK, CHUNK)], buf[p],
                         sem[p])

    def process(j, p):
        pltpu.make_async_copy(ne_hbm.at[pl.ds(base_w + j * CHUNK, CHUNK)],
                              buf[p], sem[p]).wait()
        pltpu.sync_copy(buf[p], acc.at[idx_v.at[j]], add=True)

    issue(0, 0)
    issue(1, 1)

    def pair_body(u, carry):
        process(2 * u, 0)
        issue(2 * u + 2, 0)
        process(2 * u + 1, 1)
        issue(2 * u + 3, 1)
        return carry

    n_pairs = (n_chunks - 3) // 2
    lax.fori_loop(0, n_pairs, pair_body, 0)
    j0 = 2 * n_pairs
    process(j0, 0)
    issue(j0 + 2, 0)
    process(j0 + 1, 1)
    process(j0 + 2, 0)
    plsc.subcore_barrier()
    r0 = sid * rows_per_tile
    pltpu.sync_copy(acc.at[pl.ds(r0, rows_per_tile)],
                    agg_hbm.at[cid, pl.ds(r0, rows_per_tile)])

    @pl.when(sid == 0)
    def _write_tail():
        pltpu.sync_copy(acc.at[pl.ds(NS * rows_per_tile, tail_rows)],
                        agg_hbm.at[cid, pl.ds(NS * rows_per_tile, tail_rows)])


def _tc_pre(node_feat, w_src, w_dst, block):
    n = node_feat.shape[0]
    sds = jax.ShapeDtypeStruct((n, D), jnp.float32)
    return pl.pallas_call(
        _pre_body,
        grid=(n // block,),
        in_specs=[
            pl.BlockSpec((block, D), lambda i: (i, 0)),
            pl.BlockSpec((D, D), lambda i: (0, 0)),
            pl.BlockSpec((D, D), lambda i: (0, 0)),
        ],
        out_specs=[pl.BlockSpec((block, D), lambda i: (i, 0))] * 2,
        out_shape=[sds, sds],
    )(node_feat, w_src, w_dst)


def _tc_edge(edge_feat, s, w1, b1, w2, b2, block):
    e = edge_feat.shape[0]
    sds = jax.ShapeDtypeStruct((e, D), jnp.float32)
    return pl.pallas_call(
        _edge_body,
        grid=(e // block,),
        in_specs=[
            pl.BlockSpec((block, D), lambda i: (i, 0)),
            pl.BlockSpec((block, D), lambda i: (i, 0)),
            pl.BlockSpec((D, D), lambda i: (0, 0)),
            pl.BlockSpec((1, D), lambda i: (0, 0)),
            pl.BlockSpec((D, D), lambda i: (0, 0)),
            pl.BlockSpec((1, D), lambda i: (0, 0)),
        ],
        out_specs=[pl.BlockSpec((block, D), lambda i: (i, 0))] * 2,
        out_shape=[sds, sds],
    )(edge_feat, s, w1, b1, w2, b2)


def _tc_node(node_feat, a0, a1, w1a, w1b, b1, w2, b2, block):
    n = node_feat.shape[0]
    return pl.pallas_call(
        _node_body,
        grid=(n // block,),
        in_specs=[
            pl.BlockSpec((block, D), lambda i: (i, 0)),
            pl.BlockSpec((block, D), lambda i: (i, 0)),
            pl.BlockSpec((block, D), lambda i: (i, 0)),
            pl.BlockSpec((D, D), lambda i: (0, 0)),
            pl.BlockSpec((D, D), lambda i: (0, 0)),
            pl.BlockSpec((1, D), lambda i: (0, 0)),
            pl.BlockSpec((D, D), lambda i: (0, 0)),
            pl.BlockSpec((1, D), lambda i: (0, 0)),
        ],
        out_specs=pl.BlockSpec((block, D), lambda i: (i, 0)),
        out_shape=jax.ShapeDtypeStruct((n, D), jnp.float32),
    )(node_feat, a0, a1, w1a, w1b, b1, w2, b2)


def _sc_gather(p, q, src3, dst3):
    e = src3.size
    n_chunks = (e // NW) // CHUNK
    mesh = plsc.VectorSubcoreMesh(core_axis_name="c", subcore_axis_name="s",
                                  num_cores=NC, num_subcores=NS)
    k = pl.kernel(
        functools.partial(_gather_body, e),
        out_type=jax.ShapeDtypeStruct((e, D), jnp.float32),
        mesh=mesh,
        scratch_types=[
            pltpu.VMEM((n_chunks, CHUNK), jnp.int32),
            pltpu.VMEM((n_chunks, CHUNK), jnp.int32),
            [pltpu.VMEM((CHUNK, D), jnp.float32)] * 2,
            [pltpu.VMEM((CHUNK, D), jnp.float32)] * 2,
            [pltpu.SemaphoreType.DMA] * 2,
            [pltpu.SemaphoreType.DMA] * 2,
            [pltpu.SemaphoreType.DMA] * 2,
        ],
    )
    return k(p, q, src3, dst3)


def _sc_scatter(ne, dst3, n_nodes):
    e = dst3.size
    n_chunks = (e // NW) // CHUNK
    zrows = ((n_nodes // NS) // 8 * 8) // 6
    mesh = plsc.VectorSubcoreMesh(core_axis_name="c", subcore_axis_name="s",
                                  num_cores=NC, num_subcores=NS)
    k = pl.kernel(
        functools.partial(_scatter_body, n_nodes, e),
        out_type=jax.ShapeDtypeStruct((NC, n_nodes, D), jnp.float32),
        mesh=mesh,
        scratch_types=[
            pltpu.VMEM((n_chunks, CHUNK), jnp.int32),
            [pltpu.VMEM((CHUNK, D), jnp.float32)] * 2,
            pltpu.VMEM((zrows, D), jnp.float32),
            pltpu.VMEM_SHARED((n_nodes, D), jnp.float32),
            [pltpu.SemaphoreType.DMA] * 2,
        ],
    )
    return k(ne, dst3)


def kernel(node_feat, edge_feat, edge_index, We1, be1, We2, be2, Wn1, bn1,
           Wn2, bn2):
    n = node_feat.shape[0]
    e = edge_feat.shape[0]
    n_chunks = (e // NW) // CHUNK
    src3 = edge_index[0].astype(jnp.int32).reshape(NW, n_chunks, CHUNK)
    dst3 = edge_index[1].astype(jnp.int32).reshape(NW, n_chunks, CHUNK)
    w_e = We1[:D]
    w_s = We1[D:2 * D]
    w_d = We1[2 * D:]
    w1a = Wn1[:D]
    w1b = Wn1[D:]
    be1_2d = be1.reshape(1, D)
    be2_2d = be2.reshape(1, D)
    bn1_2d = bn1.reshape(1, D)
    bn2_2d = bn2.reshape(1, D)

    p, q = _tc_pre(node_feat, w_s, w_d, block=2000)
    s = _sc_gather(p, q, src3, dst3)
    ne, edge_out = _tc_edge(edge_feat, s, w_e, be1_2d, We2, be2_2d, block=2000)
    aggp = _sc_scatter(ne, dst3, n)
    node_out = _tc_node(node_feat, aggp[0], aggp[1], w1a, w1b, bn1_2d, Wn2,
                        bn2_2d, block=2000)
    return node_out, edge_out
